# baseline (speedup 1.0000x reference)
import jax
import jax.numpy as jnp
from jax.experimental import pallas as pl
from jax.experimental.pallas import tpu as pltpu

N_REGION = 100000
N_SLOT = 8
HIDDEN = 64
ROW = N_SLOT * HIDDEN
NCHUNK = 10
CHUNK = N_REGION // NCHUNK


def _body(rg_ref, mem_hbm, oemb_ref, w1_ref, w2_ref, sel_ref, selt_ref,
          out_hbm, win_v, sems, rsem):
    rg = rg_ref[0]
    bulk = [
        pltpu.make_async_copy(
            mem_hbm.at[pl.ds(c * CHUNK, CHUNK)],
            out_hbm.at[pl.ds(c * CHUNK, CHUNK)],
            sems.at[c])
        for c in range(NCHUNK)
    ]
    for cp in bulk:
        cp.start()
    j = rg % 8
    base = pl.multiple_of(rg - j, 8)
    win_cp = pltpu.make_async_copy(mem_hbm.at[pl.ds(base, 8)], win_v, rsem)
    win_cp.start()
    win_cp.wait()
    win = win_v[...]
    ids = jax.lax.broadcasted_iota(jnp.int32, (8, 1), 0)
    mask = ids == j
    row = jnp.sum(jnp.where(mask, win, 0.0), axis=0, keepdims=True)
    c0 = jax.lax.dot(oemb_ref[...] * w1_ref[...], sel_ref[...],
                     preferred_element_type=jnp.float32)
    d = jax.lax.dot(row * w2_ref[...], sel_ref[...],
                    preferred_element_type=jnp.float32)
    g = jax.nn.sigmoid(c0 + d)
    ge = jax.lax.dot(g, selt_ref[...],
                     preferred_element_type=jnp.float32)
    new_row = row * (1.0 - ge) + oemb_ref[...] * ge
    win_v[...] = jnp.where(mask, new_row, win)
    for cp in bulk:
        cp.wait()
    out_cp = pltpu.make_async_copy(win_v, out_hbm.at[pl.ds(base, 8)], rsem)
    out_cp.start()
    out_cp.wait()


def kernel(memory, o_emb, W_fg, o_rg):
    mem2d = memory.reshape(N_REGION, ROW)
    oemb512 = jnp.tile(o_emb, N_SLOT).reshape(1, ROW)
    w1_512 = jnp.tile(W_fg[0, :HIDDEN], N_SLOT).reshape(1, ROW)
    w2_512 = jnp.tile(W_fg[0, HIDDEN:], N_SLOT).reshape(1, ROW)
    sel = (jnp.arange(ROW, dtype=jnp.int32)[:, None] // HIDDEN
           == jnp.arange(N_SLOT, dtype=jnp.int32)[None, :]).astype(jnp.float32)
    rg = jnp.asarray(o_rg, jnp.int32).reshape((1,))

    out = pl.pallas_call(
        _body,
        in_specs=[
            pl.BlockSpec(memory_space=pltpu.MemorySpace.SMEM),
            pl.BlockSpec(memory_space=pltpu.MemorySpace.HBM),
            pl.BlockSpec(memory_space=pltpu.MemorySpace.VMEM),
            pl.BlockSpec(memory_space=pltpu.MemorySpace.VMEM),
            pl.BlockSpec(memory_space=pltpu.MemorySpace.VMEM),
            pl.BlockSpec(memory_space=pltpu.MemorySpace.VMEM),
            pl.BlockSpec(memory_space=pltpu.MemorySpace.VMEM),
        ],
        out_specs=pl.BlockSpec(memory_space=pltpu.MemorySpace.HBM),
        out_shape=jax.ShapeDtypeStruct((N_REGION, ROW), jnp.float32),
        scratch_shapes=[
            pltpu.VMEM((8, ROW), jnp.float32),
            pltpu.SemaphoreType.DMA((NCHUNK,)),
            pltpu.SemaphoreType.DMA,
        ],
    )(rg, mem2d, oemb512, w1_512, w2_512, sel, sel.T)
    return out.reshape(N_REGION, N_SLOT, HIDDEN)


# speedup vs baseline: 13.4974x; 13.4974x over previous
import functools

import jax
import jax.numpy as jnp
from jax.experimental import pallas as pl
from jax.experimental.pallas import tpu as pltpu

N_REGION = 100000
N_SLOT = 8
HIDDEN = 64
ROW = N_SLOT * HIDDEN
BLOCK = 2000


def _body(rg_ref, mem_ref, oemb_ref, w1_ref, w2_ref, sel_ref, selt_ref,
          out_ref):
    i = pl.program_id(0)
    rg = rg_ref[0]

    @pl.when(i != rg // BLOCK)
    def _copy():
        out_ref[...] = mem_ref[...]

    @pl.when(i == rg // BLOCK)
    def _update():
        x = mem_ref[...]
        local = rg % BLOCK
        ids = jax.lax.broadcasted_iota(jnp.int32, (BLOCK, 1), 0)
        row_mask = ids == local
        row = jnp.sum(jnp.where(row_mask, x, 0.0), axis=0, keepdims=True)
        oemb = oemb_ref[...]
        c0 = jax.lax.dot(oemb * w1_ref[...], sel_ref[...],
                         preferred_element_type=jnp.float32)
        d = jax.lax.dot(row * w2_ref[...], sel_ref[...],
                        preferred_element_type=jnp.float32)
        g = jax.nn.sigmoid(c0 + d)
        ge = jax.lax.dot(g, selt_ref[...],
                         preferred_element_type=jnp.float32)
        new_row = row * (1.0 - ge) + oemb * ge
        out_ref[...] = jnp.where(row_mask, new_row, x)


def kernel(memory, o_emb, W_fg, o_rg):
    mem2d = memory.reshape(N_REGION, ROW)
    oemb512 = jnp.tile(o_emb, N_SLOT).reshape(1, ROW)
    w1_512 = jnp.tile(W_fg[0, :HIDDEN], N_SLOT).reshape(1, ROW)
    w2_512 = jnp.tile(W_fg[0, HIDDEN:], N_SLOT).reshape(1, ROW)
    sel = (jnp.arange(ROW, dtype=jnp.int32)[:, None] // HIDDEN
           == jnp.arange(N_SLOT, dtype=jnp.int32)[None, :]).astype(jnp.float32)
    rg = jnp.asarray(o_rg, jnp.int32).reshape((1,))

    nb = N_REGION // BLOCK
    out = pl.pallas_call(
        _body,
        grid_spec=pltpu.PrefetchScalarGridSpec(
            num_scalar_prefetch=1,
            grid=(nb,),
            in_specs=[
                pl.BlockSpec((BLOCK, ROW), lambda i, rg: (i, 0)),
                pl.BlockSpec((1, ROW), lambda i, rg: (0, 0)),
                pl.BlockSpec((1, ROW), lambda i, rg: (0, 0)),
                pl.BlockSpec((1, ROW), lambda i, rg: (0, 0)),
                pl.BlockSpec((ROW, N_SLOT), lambda i, rg: (0, 0)),
                pl.BlockSpec((N_SLOT, ROW), lambda i, rg: (0, 0)),
            ],
            out_specs=pl.BlockSpec((BLOCK, ROW), lambda i, rg: (i, 0)),
        ),
        out_shape=jax.ShapeDtypeStruct((N_REGION, ROW), jnp.float32),
    )(rg, mem2d, oemb512, w1_512, w2_512, sel, sel.T)
    return out.reshape(N_REGION, N_SLOT, HIDDEN)
